# baseline (speedup 1.0000x reference)
import functools
import jax
import jax.numpy as jnp
from jax.experimental import pallas as pl
from jax.experimental.pallas import tpu as pltpu

_TOKENS = 32768
_D = 1024
_E = 8
_BT = 2048


def _gate_block(x_ref, wt_ref, b_ref, vals_ref, idx_ref):
    x = x_ref[...]
    wt = wt_ref[...]
    logits = jax.lax.dot_general(
        x, wt, (((1,), (0,)), ((), ())), preferred_element_type=jnp.float32)
    lt = logits.T + b_ref[...]

    e = jax.lax.broadcasted_iota(jnp.int32, lt.shape, 0)
    m1 = jnp.max(lt, axis=0, keepdims=True)
    i1 = jnp.min(jnp.where(lt == m1, e, _E), axis=0, keepdims=True)
    masked = jnp.where(e == i1, -jnp.inf, lt)
    m2 = jnp.max(masked, axis=0, keepdims=True)
    i2 = jnp.min(jnp.where(masked == m2, e, _E), axis=0, keepdims=True)

    vals_ref[...] = jax.nn.sigmoid(jnp.concatenate([m1, m2], axis=0)).T
    idx_ref[...] = jnp.concatenate([i1, i2], axis=0).T


def kernel(x, W, b):
    wt = W.T
    b2 = b.reshape(_E, 1)
    grid = (_TOKENS // _BT,)
    vals_t, idx_t = pl.pallas_call(
        _gate_block,
        grid=grid,
        in_specs=[
            pl.BlockSpec((_BT, _D), lambda i: (i, 0)),
            pl.BlockSpec((_D, _E), lambda i: (0, 0)),
            pl.BlockSpec((_E, 1), lambda i: (0, 0)),
        ],
        out_specs=[
            pl.BlockSpec((_BT, 2), lambda i: (i, 0)),
            pl.BlockSpec((_BT, 2), lambda i: (i, 0)),
        ],
        out_shape=[
            jax.ShapeDtypeStruct((_TOKENS, 2), jnp.float32),
            jax.ShapeDtypeStruct((_TOKENS, 2), jnp.int32),
        ],
        compiler_params=pltpu.CompilerParams(
            dimension_semantics=("parallel",),
        ),
    )(x, wt, b2)
    return vals_t, idx_t


# speedup vs baseline: 1.7030x; 1.7030x over previous
import functools
import jax
import jax.numpy as jnp
from jax.experimental import pallas as pl
from jax.experimental.pallas import tpu as pltpu

_TOKENS = 32768
_D = 1024
_E = 8
_BT = 2048


def _gate_block(x_ref, wt_ref, b_ref, vals_ref, idx_ref):
    x = x_ref[...]
    wt = wt_ref[...]
    logits = jax.lax.dot_general(
        x, wt, (((1,), (0,)), ((), ())), preferred_element_type=jnp.float32)
    lt = logits.T + b_ref[...]

    e = jax.lax.broadcasted_iota(jnp.int32, lt.shape, 0)
    m1 = jnp.max(lt, axis=0, keepdims=True)
    i1 = jnp.min(jnp.where(lt == m1, e, _E), axis=0, keepdims=True)
    masked = jnp.where(e == i1, -jnp.inf, lt)
    m2 = jnp.max(masked, axis=0, keepdims=True)
    i2 = jnp.min(jnp.where(masked == m2, e, _E), axis=0, keepdims=True)

    vals_ref[...] = jax.nn.sigmoid(jnp.concatenate([m1, m2], axis=0))
    idx_ref[...] = jnp.concatenate([i1, i2], axis=0)


def kernel(x, W, b):
    wt = W.T
    b2 = b.reshape(_E, 1)
    grid = (_TOKENS // _BT,)
    vals_t, idx_t = pl.pallas_call(
        _gate_block,
        grid=grid,
        in_specs=[
            pl.BlockSpec((_BT, _D), lambda i: (i, 0)),
            pl.BlockSpec((_D, _E), lambda i: (0, 0)),
            pl.BlockSpec((_E, 1), lambda i: (0, 0)),
        ],
        out_specs=[
            pl.BlockSpec((2, _BT), lambda i: (0, i)),
            pl.BlockSpec((2, _BT), lambda i: (0, i)),
        ],
        out_shape=[
            jax.ShapeDtypeStruct((2, _TOKENS), jnp.float32),
            jax.ShapeDtypeStruct((2, _TOKENS), jnp.int32),
        ],
        compiler_params=pltpu.CompilerParams(
            dimension_semantics=("parallel",),
        ),
    )(x, wt, b2)
    return vals_t.T, idx_t.T


# speedup vs baseline: 1.7994x; 1.0566x over previous
import functools
import jax
import jax.numpy as jnp
from jax.experimental import pallas as pl
from jax.experimental.pallas import tpu as pltpu

_TOKENS = 32768
_D = 1024
_E = 8
_BT = 2048


def _gate_block(x_ref, w_ref, b_ref, vals_ref, idx_ref):
    x = x_ref[...]
    w = w_ref[...]
    logits = jax.lax.dot_general(
        x, w, (((1,), (1,)), ((), ())), preferred_element_type=jnp.float32)
    lt = logits.T + b_ref[...]

    e = jax.lax.broadcasted_iota(jnp.int32, lt.shape, 0)
    m1 = jnp.max(lt, axis=0, keepdims=True)
    i1 = jnp.min(jnp.where(lt == m1, e, _E), axis=0, keepdims=True)
    masked = jnp.where(e == i1, -jnp.inf, lt)
    m2 = jnp.max(masked, axis=0, keepdims=True)
    i2 = jnp.min(jnp.where(masked == m2, e, _E), axis=0, keepdims=True)

    vals_ref[...] = jax.nn.sigmoid(jnp.concatenate([m1, m2], axis=0))
    idx_ref[...] = jnp.concatenate([i1, i2], axis=0)


def kernel(x, W, b):
    b2 = b.reshape(_E, 1)
    grid = (_TOKENS // _BT,)
    vals_t, idx_t = pl.pallas_call(
        _gate_block,
        grid=grid,
        in_specs=[
            pl.BlockSpec((_BT, _D), lambda i: (i, 0)),
            pl.BlockSpec((_E, _D), lambda i: (0, 0)),
            pl.BlockSpec((_E, 1), lambda i: (0, 0)),
        ],
        out_specs=[
            pl.BlockSpec((2, _BT), lambda i: (0, i)),
            pl.BlockSpec((2, _BT), lambda i: (0, i)),
        ],
        out_shape=[
            jax.ShapeDtypeStruct((2, _TOKENS), jnp.float32),
            jax.ShapeDtypeStruct((2, _TOKENS), jnp.int32),
        ],
        compiler_params=pltpu.CompilerParams(
            dimension_semantics=("parallel",),
        ),
    )(x, W, b2)
    return vals_t.T, idx_t.T
